# Initial kernel scaffold; baseline (speedup 1.0000x reference)
import jax
import jax.numpy as jnp
from jax.experimental import pallas as pl


def kernel(x, memory_keys, memory_values, params):
    raise NotImplementedError("write your pallas kernel here")


# speedup vs baseline: 1.2082x; 1.2082x over previous
import functools

import jax
import jax.numpy as jnp
from jax import lax
from jax.experimental import pallas as pl
from jax.experimental.pallas import tpu as pltpu
from jax.experimental.pallas import tpu_sc as plsc

B = 1024
S = 64
D = 6
DM = 64
PL = 24
TOPK = 5
NL = 3
NH = 4
HD = DM // NH
M = 100000

_TILE = 2048
_NM = (M + _TILE - 1) // _TILE
_BBLK = 256
_NB = B // _BBLK
_NEG = -1e30


def _enc_ln(x, g, b, eps=1e-5):
    mu = x.mean(axis=-1, keepdims=True)
    var = ((x - mu) ** 2).mean(axis=-1, keepdims=True)
    return (x - mu) / jnp.sqrt(var + eps) * g + b


def _enc_gpt2(h, p):
    Bq, T, _ = h.shape
    mask = jnp.tril(jnp.ones((T, T), dtype=bool))
    for l in range(NL):
        hn = _enc_ln(h, p["ln1_g"][l], p["ln1_b"][l])
        qkv = hn @ p["attn_w"][l] + p["attn_b"][l]
        q, k, v = jnp.split(qkv, 3, axis=-1)
        q = q.reshape(Bq, T, NH, HD).transpose(0, 2, 1, 3)
        k = k.reshape(Bq, T, NH, HD).transpose(0, 2, 1, 3)
        v = v.reshape(Bq, T, NH, HD).transpose(0, 2, 1, 3)
        att = (q @ k.transpose(0, 1, 3, 2)) / jnp.sqrt(jnp.float32(HD))
        att = jnp.where(mask[None, None], att, jnp.finfo(jnp.float32).min)
        att = jax.nn.softmax(att, axis=-1)
        o = (att @ v).transpose(0, 2, 1, 3).reshape(Bq, T, DM)
        h = h + o @ p["aproj_w"][l] + p["aproj_b"][l]
        hn = _enc_ln(h, p["ln2_g"][l], p["ln2_b"][l])
        ff = jax.nn.gelu(hn @ p["fc_w"][l] + p["fc_b"][l], approximate=True)
        h = h + ff @ p["mproj_w"][l] + p["mproj_b"][l]
    return _enc_ln(h, p["lnf_g"], p["lnf_b"])


def _topk_body(q_ref, keys_ref, w_ref, idx_ref, rv_ref, ri_ref):
    j = pl.program_id(1)

    @pl.when(j == 0)
    def _init():
        rv_ref[...] = jnp.full((_BBLK, 8), _NEG, jnp.float32)
        ri_ref[...] = jnp.zeros((_BBLK, 8), jnp.int32)

    keys = keys_ref[...]
    rn = lax.rsqrt(jnp.maximum(jnp.sum(keys * keys, axis=1, keepdims=True),
                               1e-24))
    keysn = keys * rn
    qb = q_ref[...]
    sim = lax.dot_general(qb, keysn, (((1,), (1,)), ((), ())),
                          preferred_element_type=jnp.float32)
    col = j * _TILE + lax.broadcasted_iota(jnp.int32, (_BBLK, _TILE), 1)
    sim = jnp.where(col < M, sim, _NEG)

    aug_v = jnp.concatenate([rv_ref[...], sim], axis=1)
    aug_i = jnp.concatenate([ri_ref[...], col], axis=1)
    W = _TILE + 8
    lane = lax.broadcasted_iota(jnp.int32, (_BBLK, W), 1)
    vals = aug_v
    tv, ti = [], []
    for _ in range(TOPK):
        m = jnp.max(vals, axis=1, keepdims=True)
        pos = jnp.min(jnp.where(vals == m, lane, jnp.int32(2 ** 30)),
                      axis=1, keepdims=True)
        sel = lane == pos
        ik = jnp.sum(jnp.where(sel, aug_i, 0), axis=1, keepdims=True)
        vals = jnp.where(sel, _NEG, vals)
        tv.append(m)
        ti.append(ik)

    fpad = jnp.full((_BBLK, 3), _NEG, jnp.float32)
    ipad = jnp.zeros((_BBLK, 3), jnp.int32)
    rv_ref[...] = jnp.concatenate(tv + [fpad], axis=1)
    ri_ref[...] = jnp.concatenate(ti + [ipad], axis=1)

    @pl.when(j == _NM - 1)
    def _fin():
        v = jnp.concatenate(tv, axis=1)
        e = jnp.exp(v - tv[0])
        w5 = e / jnp.sum(e, axis=1, keepdims=True)
        w_ref[...] = jnp.concatenate([w5, jnp.zeros((_BBLK, 3), jnp.float32)],
                                     axis=1)
        idx_ref[...] = jnp.concatenate(ti + [ipad], axis=1)


def _run_topk(qn, keys):
    return pl.pallas_call(
        _topk_body,
        grid=(_NB, _NM),
        in_specs=[
            pl.BlockSpec((_BBLK, DM), lambda i, j: (i, 0)),
            pl.BlockSpec((_TILE, DM), lambda i, j: (j, 0)),
        ],
        out_specs=[
            pl.BlockSpec((_BBLK, 8), lambda i, j: (i, 0)),
            pl.BlockSpec((_BBLK, 8), lambda i, j: (i, 0)),
        ],
        out_shape=[
            jax.ShapeDtypeStruct((B, 8), jnp.float32),
            jax.ShapeDtypeStruct((B, 8), jnp.int32),
        ],
        scratch_shapes=[
            pltpu.VMEM((_BBLK, 8), jnp.float32),
            pltpu.VMEM((_BBLK, 8), jnp.int32),
        ],
    )(qn, keys)


_NROWS = B * TOPK
_RW = PL * D
_RPW = _NROWS // 32
_CH = _RPW // 2


def _sc_gather(mv2d, idx2d):
    mesh = plsc.VectorSubcoreMesh(core_axis_name="c", subcore_axis_name="s")

    @functools.partial(
        pl.kernel, mesh=mesh,
        out_type=jax.ShapeDtypeStruct((_NROWS, _RW), jnp.float32),
        compiler_params=pltpu.CompilerParams(use_tc_tiling_on_sc=False),
        scratch_types=[
            pltpu.VMEM((_CH,), jnp.int32),
            pltpu.VMEM((_CH,), jnp.int32),
            pltpu.VMEM((_RPW, _RW), jnp.float32),
            pltpu.SemaphoreType.DMA,
        ],
    )
    def k(mv_hbm, idx_hbm, out_hbm, idx_a, idx_b, rows, sem):
        wid = lax.axis_index("s") * 2 + lax.axis_index("c")
        pltpu.sync_copy(idx_hbm.at[2 * wid], idx_a)
        pltpu.sync_copy(idx_hbm.at[2 * wid + 1], idx_b)
        pltpu.async_copy(mv_hbm.at[idx_a], rows.at[pl.ds(0, _CH)], sem).wait()
        pltpu.async_copy(mv_hbm.at[idx_b], rows.at[pl.ds(_CH, _CH)], sem).wait()
        pltpu.sync_copy(rows, out_hbm.at[pl.ds(wid * _RPW, _RPW)])

    return k(mv2d, idx2d)


def _fuse_body(vals_ref, w_ref, basea_ref, out_ref):
    acc = w_ref[:, 0:1] * vals_ref[:, 0, :]
    for k in range(1, TOPK):
        acc = acc + w_ref[:, k:k + 1] * vals_ref[:, k, :]
    out_ref[...] = basea_ref[...] + acc


def _run_fuse(vals, w, base_a):
    return pl.pallas_call(
        _fuse_body,
        grid=(_NB,),
        in_specs=[
            pl.BlockSpec((_BBLK, TOPK, _RW), lambda i: (i, 0, 0)),
            pl.BlockSpec((_BBLK, 8), lambda i: (i, 0)),
            pl.BlockSpec((_BBLK, _RW), lambda i: (i, 0)),
        ],
        out_specs=pl.BlockSpec((_BBLK, _RW), lambda i: (i, 0)),
        out_shape=jax.ShapeDtypeStruct((B, _RW), jnp.float32),
    )(vals, w, base_a)


def kernel(x, memory_keys, memory_values, params):
    p = params
    x_emb = x @ p["in_w"] + p["in_b"]
    h = _enc_gpt2(x_emb + p["wpe"][:S][None], p)
    base = h.reshape(B, S * DM) @ p["out_w"] + p["out_b"]

    qe = (x @ p["re1_w"] + p["re1_b"]).reshape(B, S * DM) @ p["re2_w"] + p["re2_b"]
    qn = qe * lax.rsqrt(jnp.maximum(jnp.sum(qe * qe, axis=1, keepdims=True),
                                    jnp.float32(1e-24)))

    w8, idx8 = _run_topk(qn, memory_keys)

    idx2d = idx8[:, :TOPK].reshape(64, _CH)
    vals = _sc_gather(memory_values.reshape(M, _RW), idx2d)

    alpha = jax.nn.sigmoid(p["fusion_w"])
    base_a = alpha * base
    w_scaled = (1.0 - alpha) * w8
    out = _run_fuse(vals.reshape(B, TOPK, _RW), w_scaled, base_a)
    return out.reshape(B, PL, D)


# speedup vs baseline: 1.2233x; 1.0125x over previous
import functools

import jax
import jax.numpy as jnp
from jax import lax
from jax.experimental import pallas as pl
from jax.experimental.pallas import tpu as pltpu
from jax.experimental.pallas import tpu_sc as plsc

B = 1024
S = 64
D = 6
DM = 64
PL = 24
TOPK = 5
NL = 3
NH = 4
HD = DM // NH
M = 100000

_TILE = 2048
_NM = (M + _TILE - 1) // _TILE
_BBLK = 256
_NB = B // _BBLK
_NEG = -1e30


def _enc_ln(x, g, b, eps=1e-5):
    mu = x.mean(axis=-1, keepdims=True)
    var = ((x - mu) ** 2).mean(axis=-1, keepdims=True)
    return (x - mu) / jnp.sqrt(var + eps) * g + b


def _enc_gpt2(h, p):
    Bq, T, _ = h.shape
    mask = jnp.tril(jnp.ones((T, T), dtype=bool))
    for l in range(NL):
        hn = _enc_ln(h, p["ln1_g"][l], p["ln1_b"][l])
        qkv = hn @ p["attn_w"][l] + p["attn_b"][l]
        q, k, v = jnp.split(qkv, 3, axis=-1)
        q = q.reshape(Bq, T, NH, HD).transpose(0, 2, 1, 3)
        k = k.reshape(Bq, T, NH, HD).transpose(0, 2, 1, 3)
        v = v.reshape(Bq, T, NH, HD).transpose(0, 2, 1, 3)
        att = (q @ k.transpose(0, 1, 3, 2)) / jnp.sqrt(jnp.float32(HD))
        att = jnp.where(mask[None, None], att, jnp.finfo(jnp.float32).min)
        att = jax.nn.softmax(att, axis=-1)
        o = (att @ v).transpose(0, 2, 1, 3).reshape(Bq, T, DM)
        h = h + o @ p["aproj_w"][l] + p["aproj_b"][l]
        hn = _enc_ln(h, p["ln2_g"][l], p["ln2_b"][l])
        ff = jax.nn.gelu(hn @ p["fc_w"][l] + p["fc_b"][l], approximate=True)
        h = h + ff @ p["mproj_w"][l] + p["mproj_b"][l]
    return _enc_ln(h, p["lnf_g"], p["lnf_b"])


def _topk_body(q_ref, keys_ref, w_ref, idx_ref, rv_ref, ri_ref):
    j = pl.program_id(1)

    @pl.when(j == 0)
    def _init():
        rv_ref[...] = jnp.full((_BBLK, 8), _NEG, jnp.float32)
        ri_ref[...] = jnp.zeros((_BBLK, 8), jnp.int32)

    keys = keys_ref[...]
    rn = lax.rsqrt(jnp.maximum(jnp.sum(keys * keys, axis=1, keepdims=True),
                               1e-24))
    keysn = keys * rn
    qb = q_ref[...]
    sim = lax.dot_general(qb, keysn, (((1,), (1,)), ((), ())),
                          preferred_element_type=jnp.float32)
    lane = lax.broadcasted_iota(jnp.int32, (_BBLK, _TILE), 1)
    sim = jnp.where(j * _TILE + lane < M, sim, _NEG)

    m0 = jnp.max(sim, axis=1, keepdims=True)

    def _cond(c):
        _, m, rv, _ = c
        return jnp.any(m > rv[:, 4:5])

    def _body(c):
        vals, m, rv, ri = c
        pos = jnp.min(jnp.where(vals == m, lane, jnp.int32(2 ** 30)),
                      axis=1, keepdims=True)
        vals = jnp.where(lane == pos, _NEG, vals)
        pv, pi = m, j * _TILE + pos
        nv, ni = [], []
        for k in range(TOPK):
            rk = rv[:, k:k + 1]
            ik = ri[:, k:k + 1]
            up = pv > rk
            nv.append(jnp.where(up, pv, rk))
            ni.append(jnp.where(up, pi, ik))
            pv = jnp.where(up, rk, pv)
            pi = jnp.where(up, ik, pi)
        rv = jnp.concatenate(nv + [rv[:, TOPK:]], axis=1)
        ri = jnp.concatenate(ni + [ri[:, TOPK:]], axis=1)
        m = jnp.max(vals, axis=1, keepdims=True)
        return vals, m, rv, ri

    _, _, rv, ri = lax.while_loop(_cond, _body,
                                  (sim, m0, rv_ref[...], ri_ref[...]))
    rv_ref[...] = rv
    ri_ref[...] = ri

    @pl.when(j == _NM - 1)
    def _fin():
        v = rv[:, :TOPK]
        e = jnp.exp(v - rv[:, 0:1])
        w5 = e / jnp.sum(e, axis=1, keepdims=True)
        w_ref[...] = jnp.concatenate([w5, jnp.zeros((_BBLK, 3), jnp.float32)],
                                     axis=1)
        idx_ref[...] = ri


def _run_topk(qn, keys):
    return pl.pallas_call(
        _topk_body,
        grid=(_NB, _NM),
        in_specs=[
            pl.BlockSpec((_BBLK, DM), lambda i, j: (i, 0)),
            pl.BlockSpec((_TILE, DM), lambda i, j: (j, 0)),
        ],
        out_specs=[
            pl.BlockSpec((_BBLK, 8), lambda i, j: (i, 0)),
            pl.BlockSpec((_BBLK, 8), lambda i, j: (i, 0)),
        ],
        out_shape=[
            jax.ShapeDtypeStruct((B, 8), jnp.float32),
            jax.ShapeDtypeStruct((B, 8), jnp.int32),
        ],
        scratch_shapes=[
            pltpu.VMEM((_BBLK, 8), jnp.float32),
            pltpu.VMEM((_BBLK, 8), jnp.int32),
        ],
    )(qn, keys)


_NROWS = B * TOPK
_RW = PL * D
_RPW = _NROWS // 32
_CH = _RPW // 2


def _sc_gather(mv2d, idx2d):
    mesh = plsc.VectorSubcoreMesh(core_axis_name="c", subcore_axis_name="s")

    @functools.partial(
        pl.kernel, mesh=mesh,
        out_type=jax.ShapeDtypeStruct((_NROWS, _RW), jnp.float32),
        compiler_params=pltpu.CompilerParams(use_tc_tiling_on_sc=False),
        scratch_types=[
            pltpu.VMEM((_CH,), jnp.int32),
            pltpu.VMEM((_CH,), jnp.int32),
            pltpu.VMEM((_RPW, _RW), jnp.float32),
            pltpu.SemaphoreType.DMA,
        ],
    )
    def k(mv_hbm, idx_hbm, out_hbm, idx_a, idx_b, rows, sem):
        wid = lax.axis_index("s") * 2 + lax.axis_index("c")
        pltpu.sync_copy(idx_hbm.at[2 * wid], idx_a)
        pltpu.sync_copy(idx_hbm.at[2 * wid + 1], idx_b)
        pltpu.async_copy(mv_hbm.at[idx_a], rows.at[pl.ds(0, _CH)], sem).wait()
        pltpu.async_copy(mv_hbm.at[idx_b], rows.at[pl.ds(_CH, _CH)], sem).wait()
        pltpu.sync_copy(rows, out_hbm.at[pl.ds(wid * _RPW, _RPW)])

    return k(mv2d, idx2d)


def _fuse_body(vals_ref, w_ref, basea_ref, out_ref):
    acc = w_ref[:, 0:1] * vals_ref[:, 0, :]
    for k in range(1, TOPK):
        acc = acc + w_ref[:, k:k + 1] * vals_ref[:, k, :]
    out_ref[...] = basea_ref[...] + acc


def _run_fuse(vals, w, base_a):
    return pl.pallas_call(
        _fuse_body,
        grid=(_NB,),
        in_specs=[
            pl.BlockSpec((_BBLK, TOPK, _RW), lambda i: (i, 0, 0)),
            pl.BlockSpec((_BBLK, 8), lambda i: (i, 0)),
            pl.BlockSpec((_BBLK, _RW), lambda i: (i, 0)),
        ],
        out_specs=pl.BlockSpec((_BBLK, _RW), lambda i: (i, 0)),
        out_shape=jax.ShapeDtypeStruct((B, _RW), jnp.float32),
    )(vals, w, base_a)


def kernel(x, memory_keys, memory_values, params):
    p = params
    x_emb = x @ p["in_w"] + p["in_b"]
    h = _enc_gpt2(x_emb + p["wpe"][:S][None], p)
    base = h.reshape(B, S * DM) @ p["out_w"] + p["out_b"]

    qe = (x @ p["re1_w"] + p["re1_b"]).reshape(B, S * DM) @ p["re2_w"] + p["re2_b"]
    qn = qe * lax.rsqrt(jnp.maximum(jnp.sum(qe * qe, axis=1, keepdims=True),
                                    jnp.float32(1e-24)))

    w8, idx8 = _run_topk(qn, memory_keys)

    idx2d = idx8[:, :TOPK].reshape(64, _CH)
    vals = _sc_gather(memory_values.reshape(M, _RW), idx2d)

    alpha = jax.nn.sigmoid(p["fusion_w"])
    base_a = alpha * base
    w_scaled = (1.0 - alpha) * w8
    out = _run_fuse(vals.reshape(B, TOPK, _RW), w_scaled, base_a)
    return out.reshape(B, PL, D)


# speedup vs baseline: 1.2971x; 1.0604x over previous
import functools

import jax
import jax.numpy as jnp
from jax import lax
from jax.experimental import pallas as pl
from jax.experimental.pallas import tpu as pltpu
from jax.experimental.pallas import tpu_sc as plsc

B = 1024
S = 64
D = 6
DM = 64
PL = 24
TOPK = 5
NL = 3
NH = 4
HD = DM // NH
M = 100000

_TILE = 2048
_NM = (M + _TILE - 1) // _TILE
_BBLK = 256
_NB = B // _BBLK
_NEG = -1e30


def _enc_ln(x, g, b, eps=1e-5):
    mu = x.mean(axis=-1, keepdims=True)
    var = ((x - mu) ** 2).mean(axis=-1, keepdims=True)
    return (x - mu) / jnp.sqrt(var + eps) * g + b


def _enc_gpt2(h, p):
    Bq, T, _ = h.shape
    mask = jnp.tril(jnp.ones((T, T), dtype=bool))
    for l in range(NL):
        hn = _enc_ln(h, p["ln1_g"][l], p["ln1_b"][l])
        qkv = hn @ p["attn_w"][l] + p["attn_b"][l]
        q, k, v = jnp.split(qkv, 3, axis=-1)
        q = q.reshape(Bq, T, NH, HD).transpose(0, 2, 1, 3)
        k = k.reshape(Bq, T, NH, HD).transpose(0, 2, 1, 3)
        v = v.reshape(Bq, T, NH, HD).transpose(0, 2, 1, 3)
        att = (q @ k.transpose(0, 1, 3, 2)) / jnp.sqrt(jnp.float32(HD))
        att = jnp.where(mask[None, None], att, jnp.finfo(jnp.float32).min)
        att = jax.nn.softmax(att, axis=-1)
        o = (att @ v).transpose(0, 2, 1, 3).reshape(Bq, T, DM)
        h = h + o @ p["aproj_w"][l] + p["aproj_b"][l]
        hn = _enc_ln(h, p["ln2_g"][l], p["ln2_b"][l])
        ff = jax.nn.gelu(hn @ p["fc_w"][l] + p["fc_b"][l], approximate=True)
        h = h + ff @ p["mproj_w"][l] + p["mproj_b"][l]
    return _enc_ln(h, p["lnf_g"], p["lnf_b"])


def _topk_body(q_ref, keys_ref, w_ref, idx_ref, rv_ref, ri_ref):
    j = pl.program_id(1)

    @pl.when(j == 0)
    def _init():
        rv_ref[...] = jnp.full((_BBLK, 8), _NEG, jnp.float32)
        ri_ref[...] = jnp.zeros((_BBLK, 8), jnp.int32)

    keys = keys_ref[...]
    rn = lax.rsqrt(jnp.maximum(jnp.sum(keys * keys, axis=1, keepdims=True),
                               1e-24))
    keysn = keys * rn
    qb = q_ref[...]
    sim = lax.dot_general(qb, keysn, (((1,), (1,)), ((), ())),
                          preferred_element_type=jnp.float32)
    lane = lax.broadcasted_iota(jnp.int32, (_BBLK, _TILE), 1)
    sim = jnp.where(j * _TILE + lane < M, sim, _NEG)

    rv = rv_ref[...]
    ri = ri_ref[...]
    vals = sim
    for _ in range(TOPK):
        m = jnp.max(vals, axis=1, keepdims=True)
        pos = jnp.min(jnp.where(vals == m, lane, jnp.int32(2 ** 30)),
                      axis=1, keepdims=True)
        vals = jnp.where(lane == pos, _NEG, vals)
        pv = m
        pi = j * _TILE + pos
        rshift = jnp.concatenate([pv, rv[:, :TOPK - 1]], axis=1)
        ishift = jnp.concatenate([pi, ri[:, :TOPK - 1]], axis=1)
        a = jnp.minimum(rshift, pv)
        a_idx = jnp.where(pv <= rshift, pi, ishift)
        up = a > rv[:, :TOPK]
        rv5 = jnp.where(up, a, rv[:, :TOPK])
        ri5 = jnp.where(up, a_idx, ri[:, :TOPK])
        rv = jnp.concatenate([rv5, rv[:, TOPK:]], axis=1)
        ri = jnp.concatenate([ri5, ri[:, TOPK:]], axis=1)
    rv_ref[...] = rv
    ri_ref[...] = ri

    @pl.when(j == _NM - 1)
    def _fin():
        v = rv[:, :TOPK]
        e = jnp.exp(v - rv[:, 0:1])
        w5 = e / jnp.sum(e, axis=1, keepdims=True)
        w_ref[...] = jnp.concatenate([w5, jnp.zeros((_BBLK, 3), jnp.float32)],
                                     axis=1)
        idx_ref[...] = ri


def _run_topk(qn, keys):
    return pl.pallas_call(
        _topk_body,
        grid=(_NB, _NM),
        in_specs=[
            pl.BlockSpec((_BBLK, DM), lambda i, j: (i, 0)),
            pl.BlockSpec((_TILE, DM), lambda i, j: (j, 0)),
        ],
        out_specs=[
            pl.BlockSpec((_BBLK, 8), lambda i, j: (i, 0)),
            pl.BlockSpec((_BBLK, 8), lambda i, j: (i, 0)),
        ],
        out_shape=[
            jax.ShapeDtypeStruct((B, 8), jnp.float32),
            jax.ShapeDtypeStruct((B, 8), jnp.int32),
        ],
        scratch_shapes=[
            pltpu.VMEM((_BBLK, 8), jnp.float32),
            pltpu.VMEM((_BBLK, 8), jnp.int32),
        ],
    )(qn, keys)


_NROWS = B * TOPK
_RW = PL * D
_RPW = _NROWS // 32
_CH = _RPW // 2


def _sc_gather(mv2d, idx2d):
    mesh = plsc.VectorSubcoreMesh(core_axis_name="c", subcore_axis_name="s")

    @functools.partial(
        pl.kernel, mesh=mesh,
        out_type=jax.ShapeDtypeStruct((_NROWS, _RW), jnp.float32),
        compiler_params=pltpu.CompilerParams(use_tc_tiling_on_sc=False),
        scratch_types=[
            pltpu.VMEM((_CH,), jnp.int32),
            pltpu.VMEM((_CH,), jnp.int32),
            pltpu.VMEM((_RPW, _RW), jnp.float32),
            pltpu.SemaphoreType.DMA,
        ],
    )
    def k(mv_hbm, idx_hbm, out_hbm, idx_a, idx_b, rows, sem):
        wid = lax.axis_index("s") * 2 + lax.axis_index("c")
        pltpu.sync_copy(idx_hbm.at[2 * wid], idx_a)
        pltpu.sync_copy(idx_hbm.at[2 * wid + 1], idx_b)
        pltpu.async_copy(mv_hbm.at[idx_a], rows.at[pl.ds(0, _CH)], sem).wait()
        pltpu.async_copy(mv_hbm.at[idx_b], rows.at[pl.ds(_CH, _CH)], sem).wait()
        pltpu.sync_copy(rows, out_hbm.at[pl.ds(wid * _RPW, _RPW)])

    return k(mv2d, idx2d)


def _fuse_body(vals_ref, w_ref, basea_ref, out_ref):
    acc = w_ref[:, 0:1] * vals_ref[:, 0, :]
    for k in range(1, TOPK):
        acc = acc + w_ref[:, k:k + 1] * vals_ref[:, k, :]
    out_ref[...] = basea_ref[...] + acc


def _run_fuse(vals, w, base_a):
    return pl.pallas_call(
        _fuse_body,
        grid=(_NB,),
        in_specs=[
            pl.BlockSpec((_BBLK, TOPK, _RW), lambda i: (i, 0, 0)),
            pl.BlockSpec((_BBLK, 8), lambda i: (i, 0)),
            pl.BlockSpec((_BBLK, _RW), lambda i: (i, 0)),
        ],
        out_specs=pl.BlockSpec((_BBLK, _RW), lambda i: (i, 0)),
        out_shape=jax.ShapeDtypeStruct((B, _RW), jnp.float32),
    )(vals, w, base_a)


def kernel(x, memory_keys, memory_values, params):
    p = params
    qe = (x @ p["re1_w"] + p["re1_b"]).reshape(B, S * DM) @ p["re2_w"] + p["re2_b"]
    qn = qe * lax.rsqrt(jnp.maximum(jnp.sum(qe * qe, axis=1, keepdims=True),
                                    jnp.float32(1e-24)))

    w8, idx8 = _run_topk(qn, memory_keys)

    idx2d = idx8[:, :TOPK].reshape(64, _CH)
    vals = _sc_gather(memory_values.reshape(M, _RW), idx2d)

    x_emb = x @ p["in_w"] + p["in_b"]
    h = _enc_gpt2(x_emb + p["wpe"][:S][None], p)
    base = h.reshape(B, S * DM) @ p["out_w"] + p["out_b"]

    alpha = jax.nn.sigmoid(p["fusion_w"])
    base_a = alpha * base
    w_scaled = (1.0 - alpha) * w8
    out = _run_fuse(vals.reshape(B, TOPK, _RW), w_scaled, base_a)
    return out.reshape(B, PL, D)
